# baseline (speedup 1.0000x reference)
import functools

import jax
import jax.numpy as jnp
from jax import lax
from jax.experimental import pallas as pl
from jax.experimental.pallas import tpu as pltpu
from jax.experimental.pallas import tpu_sc as plsc

N = 65536
NT = 16
CH = N // NT
NV = CH // 16
HV = NV // 2
D = 1024
SLAB = D // NT
KSUB = 0x3F7FFFFF
EPS = 1e-07
LN2 = 0.6931471805599453
MANT = 0x007FFFFF
ONEB = 0x3F800000

_mesh = plsc.VectorSubcoreMesh(core_axis_name="c", subcore_axis_name="s",
                               num_cores=1)
_CPARAMS = pltpu.CompilerParams(needs_layout_passes=False)


def _renorm(pm, ex):
    bits = plsc.bitcast(pm, jnp.int32)
    ex = ex + ((bits >> 23) & 0xFF) - 127
    pm = plsc.bitcast((bits & MANT) | ONEB, jnp.float32)
    return pm, ex


def _sort_body(risk_hbm, dur_hbm, ev_hbm, part_o,
               key_a, idx_a, key_b, idx_b, risk_s, ev_s,
               tilecnt_s, tilebase_s, slabsum_s, fexch_s,
               dbuf, ebuf, tmp32, chk_key, chk_idx, prior, posarr, digits,
               csbuf, hist0, hist1, tbrow, slabcnt, tbslab, ss2d, ssf,
               sbuf16, fbuf16, sem):
    t = lax.axis_index("s")
    base = t * CH
    lanes = lax.iota(jnp.int32, 16)
    zeros16 = jnp.zeros((16,), jnp.int32)

    pltpu.sync_copy(dur_hbm.at[pl.ds(base, CH)], dbuf)
    pltpu.sync_copy(ev_hbm.at[pl.ds(base, CH)], tmp32)

    def setup(v, _):
        d = dbuf[pl.ds(v * 16, 16)]
        chk_key[pl.ds(v * 16, 16)] = KSUB - plsc.bitcast(d, jnp.int32)
        chk_idx[pl.ds(v * 16, 16)] = base + v * 16 + lanes
        ebuf[pl.ds(v * 16, 16)] = tmp32[pl.ds(v * 16, 16)].astype(jnp.float32)
        return 0

    lax.fori_loop(0, NV, setup, 0)
    pltpu.sync_copy(chk_key, key_a.at[pl.ds(base, CH)])
    pltpu.sync_copy(chk_idx, idx_a.at[pl.ds(base, CH)])
    pltpu.sync_copy(ebuf, ev_s.at[pl.ds(base, CH)])
    pltpu.sync_copy(risk_hbm.at[pl.ds(base, CH)], dbuf)
    pltpu.sync_copy(dbuf, risk_s.at[pl.ds(base, CH)])

    def maxb(v, m):
        return jnp.maximum(m, dbuf[pl.ds(v * 16, 16)])

    mv = lax.fori_loop(0, NV, maxb, jnp.full((16,), -3.0e38, jnp.float32))
    fbuf16[...] = jnp.full((16,), jnp.max(mv), jnp.float32)
    pltpu.sync_copy(fbuf16, fexch_s.at[t])
    plsc.subcore_barrier()
    pltpu.sync_copy(fexch_s, ssf)
    gamma = jnp.max(plsc.load_gather(ssf, [lanes, zeros16]))

    def one_pass(shift, ckey, cidx, nkey, nidx, scatter_key):
        pltpu.sync_copy(ckey.at[pl.ds(base, CH)], chk_key)
        pltpu.sync_copy(cidx.at[pl.ds(base, CH)], chk_idx)

        def zero(i, _):
            hist0[pl.ds(i * 16, 16)] = zeros16
            hist1[pl.ds(i * 16, 16)] = zeros16
            return 0

        lax.fori_loop(0, D // 16, zero, 0)

        def hist_body(v, _):
            k0 = chk_key[pl.ds(v * 16, 16)]
            k1 = chk_key[pl.ds((HV + v) * 16, 16)]
            b0 = (k0 >> shift) & (D - 1)
            b1 = (k1 >> shift) & (D - 1)
            digits[pl.ds(v * 16, 16)] = b0
            digits[pl.ds((HV + v) * 16, 16)] = b1
            sc0, lm0 = plsc.scan_count(b0)
            sc1, lm1 = plsc.scan_count(b1)
            pv0 = plsc.load_gather(hist0, [b0])
            pv1 = plsc.load_gather(hist1, [b1])
            prior[pl.ds(v * 16, 16)] = pv0 + sc0 - 1
            prior[pl.ds((HV + v) * 16, 16)] = pv1 + sc1 - 1
            plsc.store_scatter(hist0, [b0], pv0 + sc0, mask=lm0)
            plsc.store_scatter(hist1, [b1], pv1 + sc1, mask=lm1)
            return 0

        lax.fori_loop(0, HV, hist_body, 0)

        def hsum(i, _):
            s16 = pl.ds(i * 16, 16)
            tmp32[s16] = hist0[s16] + hist1[s16]
            return 0

        lax.fori_loop(0, D // 16, hsum, 0)
        pltpu.sync_copy(tmp32.at[pl.ds(0, D)], tilecnt_s.at[t])
        plsc.subcore_barrier()

        for tt in range(NT):
            pltpu.sync_copy(tilecnt_s.at[tt, pl.ds(t * SLAB, SLAB)],
                            slabcnt.at[tt])
        tots = []
        for g in range(SLAB // 16):
            run = zeros16
            for tt in range(NT):
                v = slabcnt[tt, pl.ds(g * 16, 16)]
                tbslab[tt, pl.ds(g * 16, 16)] = run
                run = run + v
            tots.append(run)
        carry = jnp.int32(0)
        sgs = []
        for g in range(SLAB // 16):
            incl = plsc.cumsum(tots[g])
            sgs.append(incl - tots[g] + carry)
            carry = carry + jnp.sum(tots[g])
        sbuf16[...] = jnp.full((16,), carry, jnp.int32)
        pltpu.sync_copy(sbuf16, slabsum_s.at[t])
        plsc.subcore_barrier()
        pltpu.sync_copy(slabsum_s, ss2d)
        s_all = plsc.load_gather(ss2d, [lanes, zeros16])
        gs = plsc.cumsum(s_all) - s_all
        g_self = jnp.sum(jnp.where(lanes == t, gs, 0))
        for g in range(SLAB // 16):
            sg = sgs[g] + g_self
            for tt in range(NT):
                tbslab[tt, pl.ds(g * 16, 16)] = (
                    tbslab[tt, pl.ds(g * 16, 16)] + sg)
        pltpu.sync_copy(tbslab, tilebase_s.at[t])
        plsc.subcore_barrier()
        for u2 in range(NT):
            pltpu.sync_copy(tilebase_s.at[u2, t], tbrow.at[pl.ds(u2 * SLAB, SLAB)])

        def pos_body(v, _):
            b0 = digits[pl.ds(v * 16, 16)]
            b1 = digits[pl.ds((HV + v) * 16, 16)]
            tb0 = plsc.load_gather(tbrow, [b0])
            tb1 = plsc.load_gather(tbrow, [b1])
            h0 = plsc.load_gather(hist0, [b1])
            posarr[pl.ds(v * 16, 16)] = tb0 + prior[pl.ds(v * 16, 16)]
            posarr[pl.ds((HV + v) * 16, 16)] = (
                tb1 + h0 + prior[pl.ds((HV + v) * 16, 16)])
            return 0

        lax.fori_loop(0, HV, pos_body, 0)
        hs = []
        if scatter_key:
            hs.append(pltpu.async_copy(chk_key, nkey.at[posarr], sem))
        hs.append(pltpu.async_copy(chk_idx, nidx.at[posarr], sem))
        for h in hs:
            h.wait()
        plsc.subcore_barrier()

    one_pass(0, key_a, idx_a, key_b, idx_b, True)
    one_pass(10, key_b, idx_b, key_a, idx_a, True)
    one_pass(20, key_a, idx_a, key_b, idx_b, False)

    pltpu.sync_copy(idx_b.at[pl.ds(base, CH)], chk_idx)
    h1 = pltpu.async_copy(risk_s.at[chk_idx], dbuf, sem)
    h2 = pltpu.async_copy(ev_s.at[chk_idx], ebuf, sem)
    h1.wait()
    h2.wait()

    def cs_body(v, cc):
        c0, c1 = cc
        x0 = jnp.exp(dbuf[pl.ds(v * 16, 16)] - gamma)
        x1 = jnp.exp(dbuf[pl.ds((HV + v) * 16, 16)] - gamma)
        s0 = plsc.cumsum(x0) + c0
        s1 = plsc.cumsum(x1) + c1
        csbuf[pl.ds(v * 16, 16)] = s0
        csbuf[pl.ds((HV + v) * 16, 16)] = s1
        return (jnp.sum(jnp.where(lanes == 15, s0, 0.0)),
                jnp.sum(jnp.where(lanes == 15, s1, 0.0)))

    c0, c1 = lax.fori_loop(0, HV, cs_body, (jnp.float32(0), jnp.float32(0)))
    fbuf16[...] = jnp.where(lanes == 0, c0, jnp.where(lanes == 1, c1, 0.0))
    pltpu.sync_copy(fbuf16, fexch_s.at[t])
    plsc.subcore_barrier()
    pltpu.sync_copy(fexch_s, ssf)
    h0s = plsc.load_gather(ssf, [lanes, zeros16])
    h1s = plsc.load_gather(ssf, [lanes, zeros16 + 1])
    tilesum = h0s + h1s
    pref = plsc.cumsum(tilesum) - tilesum
    base0 = jnp.sum(jnp.where(lanes == t, pref, 0.0))
    base1 = base0 + c0

    def pr_body(v, st):
        pm0, ex0, pm1, ex1 = st
        cv0 = csbuf[pl.ds(v * 16, 16)] + base0
        e0 = ebuf[pl.ds(v * 16, 16)]
        y0 = jnp.where(e0 > 0.5, cv0 + EPS, 1.0)
        pm0, ex0 = _renorm(pm0 * y0, ex0)
        cv1 = csbuf[pl.ds((HV + v) * 16, 16)] + base1
        e1 = ebuf[pl.ds((HV + v) * 16, 16)]
        y1 = jnp.where(e1 > 0.5, cv1 + EPS, 1.0)
        pm1, ex1 = _renorm(pm1 * y1, ex1)
        return (pm0, ex0, pm1, ex1)

    ones_f = jnp.ones((16,), jnp.float32)
    pm0, ex0, pm1, ex1 = lax.fori_loop(
        0, HV, pr_body, (ones_f, zeros16, ones_f, zeros16))
    pm, ex = _renorm(pm0 * pm1, ex0 + ex1)

    row_specs = [pm, ex.astype(jnp.float32),
                 jnp.full((16,), gamma, jnp.float32),
                 ones_f, ones_f, ones_f, ones_f, ones_f]
    for r, val in enumerate(row_specs):
        fbuf16[...] = val
        pltpu.sync_copy(fbuf16, part_o.at[r, pl.ds(t * 16, 16)])


_sc_sort = functools.partial(
    pl.kernel,
    out_type=[jax.ShapeDtypeStruct((8, 256), jnp.float32)],
    mesh=_mesh,
    compiler_params=_CPARAMS,
    scratch_types=[
        pltpu.VMEM_SHARED((N,), jnp.int32),
        pltpu.VMEM_SHARED((N,), jnp.int32),
        pltpu.VMEM_SHARED((N,), jnp.int32),
        pltpu.VMEM_SHARED((N,), jnp.int32),
        pltpu.VMEM_SHARED((N,), jnp.float32),
        pltpu.VMEM_SHARED((N,), jnp.float32),
        pltpu.VMEM_SHARED((NT, D), jnp.int32),
        pltpu.VMEM_SHARED((NT, NT, SLAB), jnp.int32),
        pltpu.VMEM_SHARED((NT, 16), jnp.int32),
        pltpu.VMEM_SHARED((NT, 16), jnp.float32),
        pltpu.VMEM((CH,), jnp.float32),
        pltpu.VMEM((CH,), jnp.float32),
        pltpu.VMEM((CH,), jnp.int32),
        pltpu.VMEM((CH,), jnp.int32),
        pltpu.VMEM((CH,), jnp.int32),
        pltpu.VMEM((CH,), jnp.int32),
        pltpu.VMEM((CH,), jnp.int32),
        pltpu.VMEM((CH,), jnp.int32),
        pltpu.VMEM((CH,), jnp.float32),
        pltpu.VMEM((D,), jnp.int32),
        pltpu.VMEM((D,), jnp.int32),
        pltpu.VMEM((D,), jnp.int32),
        pltpu.VMEM((NT, SLAB), jnp.int32),
        pltpu.VMEM((NT, SLAB), jnp.int32),
        pltpu.VMEM((NT, 16), jnp.int32),
        pltpu.VMEM((NT, 16), jnp.float32),
        pltpu.VMEM((16,), jnp.int32),
        pltpu.VMEM((16,), jnp.float32),
        pltpu.SemaphoreType.DMA,
    ],
)(_sort_body)


def _finish_body(p_ref, risk_ref, ev_ref, out_ref):
    p = p_ref[...]
    pm = p[0:1, :]
    ex = p[1:2, :]
    gamma = jnp.max(p[2:3, :])
    e = ev_ref[...].astype(jnp.float32)
    se = jnp.sum(e)
    ser = jnp.sum(e * risk_ref[...])
    ln_p = jnp.sum(jnp.log(pm)) + jnp.sum(ex) * LN2
    loss = (ln_p + gamma * se - ser) / se
    out_ref[...] = jnp.reshape(loss, (1, 1))


def kernel(risk_pred, durations, events):
    (part,) = _sc_sort(risk_pred, durations, events)
    out = pl.pallas_call(
        _finish_body,
        out_shape=jax.ShapeDtypeStruct((1, 1), jnp.float32),
    )(part, risk_pred.reshape(512, 128), events.reshape(512, 128))
    return out[0, 0]


# speedup vs baseline: 1.0619x; 1.0619x over previous
import functools

import jax
import jax.numpy as jnp
from jax import lax
from jax.experimental import pallas as pl
from jax.experimental.pallas import tpu as pltpu
from jax.experimental.pallas import tpu_sc as plsc

N = 65536
NT = 16
CH = N // NT
NV = CH // 16
HV = NV // 2
D = 1024
SLAB = D // NT
KSUB = 0x3F7FFFFF
R = 512
C = 128
EPS = 1e-07

_mesh = plsc.VectorSubcoreMesh(core_axis_name="c", subcore_axis_name="s",
                               num_cores=1)
_CPARAMS = pltpu.CompilerParams(needs_layout_passes=False)


def _sort_body(risk_hbm, dur_hbm, ev_hbm, risk_o, ev_o,
               key_a, idx_a, key_b, idx_b, risk_s, ev_s,
               tilecnt_s, tilebase_s, slabsum_s,
               dbuf, ebuf, tmp32, chk_key, chk_idx, prior, posarr, digits,
               hist0, hist1, tbrow, slabcnt, tbslab, ss2d, sbuf16, sem):
    t = lax.axis_index("s")
    base = t * CH
    lanes = lax.iota(jnp.int32, 16)

    pltpu.sync_copy(dur_hbm.at[pl.ds(base, CH)], dbuf)
    pltpu.sync_copy(ev_hbm.at[pl.ds(base, CH)], tmp32)

    def setup(v, _):
        d = dbuf[pl.ds(v * 16, 16)]
        chk_key[pl.ds(v * 16, 16)] = KSUB - plsc.bitcast(d, jnp.int32)
        chk_idx[pl.ds(v * 16, 16)] = base + v * 16 + lanes
        ebuf[pl.ds(v * 16, 16)] = tmp32[pl.ds(v * 16, 16)].astype(jnp.float32)
        return 0

    lax.fori_loop(0, NV, setup, 0)
    pltpu.sync_copy(chk_key, key_a.at[pl.ds(base, CH)])
    pltpu.sync_copy(chk_idx, idx_a.at[pl.ds(base, CH)])
    pltpu.sync_copy(ebuf, ev_s.at[pl.ds(base, CH)])
    pltpu.sync_copy(risk_hbm.at[pl.ds(base, CH)], dbuf)
    pltpu.sync_copy(dbuf, risk_s.at[pl.ds(base, CH)])
    plsc.subcore_barrier()

    def one_pass(shift, ckey, cidx, nkey, nidx, scatter_key):
        pltpu.sync_copy(ckey.at[pl.ds(base, CH)], chk_key)
        pltpu.sync_copy(cidx.at[pl.ds(base, CH)], chk_idx)

        def zero(i, _):
            hist0[pl.ds(i * 16, 16)] = jnp.zeros((16,), jnp.int32)
            hist1[pl.ds(i * 16, 16)] = jnp.zeros((16,), jnp.int32)
            return 0

        lax.fori_loop(0, D // 16, zero, 0)

        def hist_body(v, _):
            k0 = chk_key[pl.ds(v * 16, 16)]
            k1 = chk_key[pl.ds((HV + v) * 16, 16)]
            b0 = (k0 >> shift) & (D - 1)
            b1 = (k1 >> shift) & (D - 1)
            digits[pl.ds(v * 16, 16)] = b0
            digits[pl.ds((HV + v) * 16, 16)] = b1
            sc0, lm0 = plsc.scan_count(b0)
            sc1, lm1 = plsc.scan_count(b1)
            pv0 = plsc.load_gather(hist0, [b0])
            pv1 = plsc.load_gather(hist1, [b1])
            prior[pl.ds(v * 16, 16)] = pv0 + sc0 - 1
            prior[pl.ds((HV + v) * 16, 16)] = pv1 + sc1 - 1
            plsc.store_scatter(hist0, [b0], pv0 + sc0, mask=lm0)
            plsc.store_scatter(hist1, [b1], pv1 + sc1, mask=lm1)
            return 0

        lax.fori_loop(0, HV, hist_body, 0)

        def hsum(i, _):
            s16 = pl.ds(i * 16, 16)
            tmp32[s16] = hist0[s16] + hist1[s16]
            return 0

        lax.fori_loop(0, D // 16, hsum, 0)
        pltpu.sync_copy(tmp32.at[pl.ds(0, D)], tilecnt_s.at[t])
        plsc.subcore_barrier()

        for tt in range(NT):
            pltpu.sync_copy(tilecnt_s.at[tt, pl.ds(t * SLAB, SLAB)],
                            slabcnt.at[tt])
        tots = []
        for g in range(SLAB // 16):
            run = jnp.zeros((16,), jnp.int32)
            for tt in range(NT):
                v = slabcnt[tt, pl.ds(g * 16, 16)]
                tbslab[tt, pl.ds(g * 16, 16)] = run
                run = run + v
            tots.append(run)
        carry = jnp.int32(0)
        sgs = []
        for g in range(SLAB // 16):
            incl = plsc.cumsum(tots[g])
            sgs.append(incl - tots[g] + carry)
            carry = carry + jnp.sum(tots[g])
        sbuf16[...] = jnp.full((16,), carry, jnp.int32)
        pltpu.sync_copy(sbuf16, slabsum_s.at[t])
        plsc.subcore_barrier()
        pltpu.sync_copy(slabsum_s, ss2d)
        s_all = plsc.load_gather(ss2d, [lanes, jnp.zeros((16,), jnp.int32)])
        gs = plsc.cumsum(s_all) - s_all
        g_self = jnp.sum(jnp.where(lanes == t, gs, 0))
        for g in range(SLAB // 16):
            sg = sgs[g] + g_self
            for tt in range(NT):
                tbslab[tt, pl.ds(g * 16, 16)] = (
                    tbslab[tt, pl.ds(g * 16, 16)] + sg)
        pltpu.sync_copy(tbslab, tilebase_s.at[t])
        plsc.subcore_barrier()
        for u2 in range(NT):
            pltpu.sync_copy(tilebase_s.at[u2, t], tbrow.at[pl.ds(u2 * SLAB, SLAB)])

        def pos_body(v, _):
            b0 = digits[pl.ds(v * 16, 16)]
            b1 = digits[pl.ds((HV + v) * 16, 16)]
            tb0 = plsc.load_gather(tbrow, [b0])
            tb1 = plsc.load_gather(tbrow, [b1])
            h0 = plsc.load_gather(hist0, [b1])
            posarr[pl.ds(v * 16, 16)] = tb0 + prior[pl.ds(v * 16, 16)]
            posarr[pl.ds((HV + v) * 16, 16)] = (
                tb1 + h0 + prior[pl.ds((HV + v) * 16, 16)])
            return 0

        lax.fori_loop(0, HV, pos_body, 0)
        hs = []
        if scatter_key:
            hs.append(pltpu.async_copy(chk_key, nkey.at[posarr], sem))
        hs.append(pltpu.async_copy(chk_idx, nidx.at[posarr], sem))
        for h in hs:
            h.wait()
        plsc.subcore_barrier()

    one_pass(0, key_a, idx_a, key_b, idx_b, True)
    one_pass(10, key_b, idx_b, key_a, idx_a, True)
    one_pass(20, key_a, idx_a, key_b, idx_b, False)

    pltpu.sync_copy(idx_b.at[pl.ds(base, CH)], chk_idx)
    h1 = pltpu.async_copy(risk_s.at[chk_idx], dbuf, sem)
    h2 = pltpu.async_copy(ev_s.at[chk_idx], ebuf, sem)
    h1.wait()
    h2.wait()
    pltpu.sync_copy(dbuf, risk_o.at[pl.ds(base, CH)])
    pltpu.sync_copy(ebuf, ev_o.at[pl.ds(base, CH)])


_sc_sort = functools.partial(
    pl.kernel,
    out_type=[jax.ShapeDtypeStruct((N,), jnp.float32),
              jax.ShapeDtypeStruct((N,), jnp.float32)],
    mesh=_mesh,
    compiler_params=_CPARAMS,
    scratch_types=[
        pltpu.VMEM_SHARED((N,), jnp.int32),
        pltpu.VMEM_SHARED((N,), jnp.int32),
        pltpu.VMEM_SHARED((N,), jnp.int32),
        pltpu.VMEM_SHARED((N,), jnp.int32),
        pltpu.VMEM_SHARED((N,), jnp.float32),
        pltpu.VMEM_SHARED((N,), jnp.float32),
        pltpu.VMEM_SHARED((NT, D), jnp.int32),
        pltpu.VMEM_SHARED((NT, NT, SLAB), jnp.int32),
        pltpu.VMEM_SHARED((NT, 16), jnp.int32),
        pltpu.VMEM((CH,), jnp.float32),
        pltpu.VMEM((CH,), jnp.float32),
        pltpu.VMEM((CH,), jnp.int32),
        pltpu.VMEM((CH,), jnp.int32),
        pltpu.VMEM((CH,), jnp.int32),
        pltpu.VMEM((CH,), jnp.int32),
        pltpu.VMEM((CH,), jnp.int32),
        pltpu.VMEM((CH,), jnp.int32),
        pltpu.VMEM((D,), jnp.int32),
        pltpu.VMEM((D,), jnp.int32),
        pltpu.VMEM((D,), jnp.int32),
        pltpu.VMEM((NT, SLAB), jnp.int32),
        pltpu.VMEM((NT, SLAB), jnp.int32),
        pltpu.VMEM((NT, 16), jnp.int32),
        pltpu.VMEM((16,), jnp.int32),
        pltpu.SemaphoreType.DMA,
    ],
)(_sort_body)


def _loss_body(risk_ref, ev_ref, out_ref):
    risk = risk_ref[...]
    ev = ev_ref[...]
    gamma = jnp.max(risk)
    x = jnp.exp(risk - gamma)
    jc = jax.lax.broadcasted_iota(jnp.int32, (C, C), 0)
    kc = jax.lax.broadcasted_iota(jnp.int32, (C, C), 1)
    upper = (jc <= kc).astype(jnp.float32)
    ir = jax.lax.broadcasted_iota(jnp.int32, (R, R), 0)
    kr = jax.lax.broadcasted_iota(jnp.int32, (R, R), 1)
    lower = (kr < ir).astype(jnp.float32)
    row_cs = jax.lax.dot(x, upper, precision=jax.lax.Precision.HIGHEST)
    prev_rows = jax.lax.dot(lower, x, precision=jax.lax.Precision.HIGHEST)
    prefix = jnp.sum(prev_rows, axis=1, keepdims=True)
    cs = row_cs + prefix
    lsh = jnp.log(cs + EPS) + gamma
    num = jnp.sum((risk - lsh) * ev)
    den = jnp.sum(ev)
    out_ref[...] = jnp.reshape(-num / den, (1, 1))


def kernel(risk_pred, durations, events):
    risk_s, ev_s = _sc_sort(risk_pred, durations, events)
    out = pl.pallas_call(
        _loss_body,
        out_shape=jax.ShapeDtypeStruct((1, 1), jnp.float32),
    )(risk_s.reshape(R, C), ev_s.reshape(R, C))
    return out[0, 0]


# speedup vs baseline: 1.2011x; 1.1311x over previous
import functools

import jax
import jax.numpy as jnp
from jax import lax
from jax.experimental import pallas as pl
from jax.experimental.pallas import tpu as pltpu
from jax.experimental.pallas import tpu_sc as plsc

N = 65536
NT = 16
CH = N // NT
NV = CH // 16
HV = NV // 2
D = 1024
SLAB = D // NT
KSUB = 0x3F7FFFFF
R = 512
C = 128
EPS = 1e-07

_mesh = plsc.VectorSubcoreMesh(core_axis_name="c", subcore_axis_name="s",
                               num_cores=1)
_CPARAMS = pltpu.CompilerParams(needs_layout_passes=False)


def _sort_body(risk_hbm, dur_hbm, ev_hbm, risk_o, ev_o,
               key_a, idx_a, key_b, idx_b, risk_s, ev_s, tilecnt_s,
               dbuf, ebuf, tmp32, chk_key, chk_idx, prior, posarr, digits,
               hist0, hist1, tbrow, allcnt, sem):
    t = lax.axis_index("s")
    base = t * CH
    lanes = lax.iota(jnp.int32, 16)

    pltpu.sync_copy(dur_hbm.at[pl.ds(base, CH)], dbuf)
    pltpu.sync_copy(ev_hbm.at[pl.ds(base, CH)], tmp32)

    def setup(v, _):
        d = dbuf[pl.ds(v * 16, 16)]
        chk_key[pl.ds(v * 16, 16)] = KSUB - plsc.bitcast(d, jnp.int32)
        chk_idx[pl.ds(v * 16, 16)] = base + v * 16 + lanes
        ebuf[pl.ds(v * 16, 16)] = tmp32[pl.ds(v * 16, 16)].astype(jnp.float32)
        return 0

    lax.fori_loop(0, NV, setup, 0)
    pltpu.sync_copy(chk_key, key_a.at[pl.ds(base, CH)])
    pltpu.sync_copy(chk_idx, idx_a.at[pl.ds(base, CH)])
    pltpu.sync_copy(ebuf, ev_s.at[pl.ds(base, CH)])
    pltpu.sync_copy(risk_hbm.at[pl.ds(base, CH)], dbuf)
    pltpu.sync_copy(dbuf, risk_s.at[pl.ds(base, CH)])
    plsc.subcore_barrier()

    def one_pass(shift, ckey, cidx, nkey, nidx, scatter_key, load_chunks):
        if load_chunks:
            pltpu.sync_copy(ckey.at[pl.ds(base, CH)], chk_key)
            pltpu.sync_copy(cidx.at[pl.ds(base, CH)], chk_idx)

        def zero(i, _):
            hist0[pl.ds(i * 16, 16)] = jnp.zeros((16,), jnp.int32)
            hist1[pl.ds(i * 16, 16)] = jnp.zeros((16,), jnp.int32)
            return 0

        lax.fori_loop(0, D // 16, zero, 0)

        def hist_body(v, _):
            k0 = chk_key[pl.ds(v * 16, 16)]
            k1 = chk_key[pl.ds((HV + v) * 16, 16)]
            b0 = (k0 >> shift) & (D - 1)
            b1 = (k1 >> shift) & (D - 1)
            digits[pl.ds(v * 16, 16)] = b0
            digits[pl.ds((HV + v) * 16, 16)] = b1
            sc0, lm0 = plsc.scan_count(b0)
            sc1, lm1 = plsc.scan_count(b1)
            pv0 = plsc.load_gather(hist0, [b0])
            pv1 = plsc.load_gather(hist1, [b1])
            prior[pl.ds(v * 16, 16)] = pv0 + sc0 - 1
            prior[pl.ds((HV + v) * 16, 16)] = pv1 + sc1 - 1
            plsc.store_scatter(hist0, [b0], pv0 + sc0, mask=lm0)
            plsc.store_scatter(hist1, [b1], pv1 + sc1, mask=lm1)
            return 0

        lax.fori_loop(0, HV, hist_body, 0)

        def hsum(i, _):
            s16 = pl.ds(i * 16, 16)
            tmp32[s16] = hist0[s16] + hist1[s16]
            return 0

        lax.fori_loop(0, D // 16, hsum, 0)
        pltpu.sync_copy(tmp32.at[pl.ds(0, D)], tilecnt_s.at[t])
        plsc.subcore_barrier()

        pltpu.sync_copy(tilecnt_s, allcnt)

        def exch(g, carry):
            tot = jnp.zeros((16,), jnp.int32)
            mine = jnp.zeros((16,), jnp.int32)
            for tt in range(NT):
                v = allcnt[tt, pl.ds(g * 16, 16)]
                tot = tot + v
                mine = mine + jnp.where(t > tt, v, 0)
            incl = plsc.cumsum(tot)
            tbrow[pl.ds(g * 16, 16)] = incl - tot + carry + mine
            return carry + jnp.sum(tot)

        lax.fori_loop(0, D // 16, exch, jnp.int32(0))

        def pos_body(v, _):
            b0 = digits[pl.ds(v * 16, 16)]
            b1 = digits[pl.ds((HV + v) * 16, 16)]
            tb0 = plsc.load_gather(tbrow, [b0])
            tb1 = plsc.load_gather(tbrow, [b1])
            h0 = plsc.load_gather(hist0, [b1])
            posarr[pl.ds(v * 16, 16)] = tb0 + prior[pl.ds(v * 16, 16)]
            posarr[pl.ds((HV + v) * 16, 16)] = (
                tb1 + h0 + prior[pl.ds((HV + v) * 16, 16)])
            return 0

        lax.fori_loop(0, HV, pos_body, 0)
        hs = []
        if scatter_key:
            hs.append(pltpu.async_copy(chk_key, nkey.at[posarr], sem))
        hs.append(pltpu.async_copy(chk_idx, nidx.at[posarr], sem))
        for h in hs:
            h.wait()
        plsc.subcore_barrier()

    one_pass(0, key_a, idx_a, key_b, idx_b, True, False)
    one_pass(10, key_b, idx_b, key_a, idx_a, True, True)
    one_pass(20, key_a, idx_a, key_b, idx_b, False, True)

    pltpu.sync_copy(idx_b.at[pl.ds(base, CH)], chk_idx)
    h1 = pltpu.async_copy(risk_s.at[chk_idx], dbuf, sem)
    h2 = pltpu.async_copy(ev_s.at[chk_idx], ebuf, sem)
    h1.wait()
    h2.wait()
    pltpu.sync_copy(dbuf, risk_o.at[pl.ds(base, CH)])
    pltpu.sync_copy(ebuf, ev_o.at[pl.ds(base, CH)])


_sc_sort = functools.partial(
    pl.kernel,
    out_type=[jax.ShapeDtypeStruct((N,), jnp.float32),
              jax.ShapeDtypeStruct((N,), jnp.float32)],
    mesh=_mesh,
    compiler_params=_CPARAMS,
    scratch_types=[
        pltpu.VMEM_SHARED((N,), jnp.int32),
        pltpu.VMEM_SHARED((N,), jnp.int32),
        pltpu.VMEM_SHARED((N,), jnp.int32),
        pltpu.VMEM_SHARED((N,), jnp.int32),
        pltpu.VMEM_SHARED((N,), jnp.float32),
        pltpu.VMEM_SHARED((N,), jnp.float32),
        pltpu.VMEM_SHARED((NT, D), jnp.int32),
        pltpu.VMEM((CH,), jnp.float32),
        pltpu.VMEM((CH,), jnp.float32),
        pltpu.VMEM((CH,), jnp.int32),
        pltpu.VMEM((CH,), jnp.int32),
        pltpu.VMEM((CH,), jnp.int32),
        pltpu.VMEM((CH,), jnp.int32),
        pltpu.VMEM((CH,), jnp.int32),
        pltpu.VMEM((CH,), jnp.int32),
        pltpu.VMEM((D,), jnp.int32),
        pltpu.VMEM((D,), jnp.int32),
        pltpu.VMEM((D,), jnp.int32),
        pltpu.VMEM((NT, D), jnp.int32),
        pltpu.SemaphoreType.DMA,
    ],
)(_sort_body)


def _loss_body(risk_ref, ev_ref, out_ref):
    risk = risk_ref[...]
    ev = ev_ref[...]
    gamma = jnp.max(risk)
    x = jnp.exp(risk - gamma)
    jc = jax.lax.broadcasted_iota(jnp.int32, (C, C), 0)
    kc = jax.lax.broadcasted_iota(jnp.int32, (C, C), 1)
    upper = (jc <= kc).astype(jnp.float32)
    ir = jax.lax.broadcasted_iota(jnp.int32, (R, R), 0)
    kr = jax.lax.broadcasted_iota(jnp.int32, (R, R), 1)
    lower = (kr < ir).astype(jnp.float32)
    row_cs = jax.lax.dot(x, upper, precision=jax.lax.Precision.HIGHEST)
    prev_rows = jax.lax.dot(lower, x, precision=jax.lax.Precision.HIGHEST)
    prefix = jnp.sum(prev_rows, axis=1, keepdims=True)
    cs = row_cs + prefix
    lsh = jnp.log(cs + EPS) + gamma
    num = jnp.sum((risk - lsh) * ev)
    den = jnp.sum(ev)
    out_ref[...] = jnp.reshape(-num / den, (1, 1))


def kernel(risk_pred, durations, events):
    risk_s, ev_s = _sc_sort(risk_pred, durations, events)
    out = pl.pallas_call(
        _loss_body,
        out_shape=jax.ShapeDtypeStruct((1, 1), jnp.float32),
    )(risk_s.reshape(R, C), ev_s.reshape(R, C))
    return out[0, 0]
